# baseline (speedup 1.0000x reference)
import functools

import jax
import jax.numpy as jnp
from jax import lax
from jax.experimental import pallas as pl
from jax.experimental.pallas import tpu as pltpu
from jax.experimental.pallas import tpu_sc as plsc

N = 10000
E = 320000
H = 64
DW = 16
NC = 2
NS = 16
NW = NC * NS
EPW = E // NW
CH = 200
NCHUNK = EPW // CH
NF = 10
NPW = N // NF
KS = 2
KE = 2
ZB = 200

_mesh = plsc.VectorSubcoreMesh(
    core_axis_name="c", subcore_axis_name="s", num_cores=NC, num_subcores=NS)

_sc_params = pltpu.CompilerParams(use_tc_tiling_on_sc=False)


@functools.partial(
    pl.kernel,
    out_type=jax.ShapeDtypeStruct((NC, N, DW), jnp.float32),
    mesh=_mesh,
    scratch_types=[
        pltpu.VMEM((NCHUNK, CH), jnp.int32),
        pltpu.VMEM((CH, DW), jnp.float32),
        pltpu.VMEM((NPW, DW), jnp.float32),
        pltpu.VMEM_SHARED((N, DW), jnp.float32),
    ],
    compiler_params=_sc_params,
)
def sc_deg(dst3_hbm, out_hbm, dsti, ones, zbuf, acc):
    cid = lax.axis_index("c")
    sid = lax.axis_index("s")
    wid = sid * NC + cid

    zeros = jnp.zeros((16,), jnp.float32)
    one16 = jnp.ones((16,), jnp.float32)

    def _fill(i, _):
        r = i // (DW // 16)
        k = (i % (DW // 16)) * 16
        ones[r, pl.ds(k, 16)] = one16
        return 0

    lax.fori_loop(0, CH * (DW // 16), _fill, 0)
    pltpu.sync_copy(dst3_hbm.at[wid], dsti)

    @pl.when(sid < NF)
    def _init():
        def _zero(i, _):
            r = i // (DW // 16)
            k = (i % (DW // 16)) * 16
            zbuf[r, pl.ds(k, 16)] = zeros
            return 0

        lax.fori_loop(0, NPW * (DW // 16), _zero, 0)
        pltpu.sync_copy(zbuf, acc.at[pl.ds(sid * NPW, NPW)])

    plsc.subcore_barrier()

    def _chunk(i, _):
        pltpu.sync_copy(ones, acc.at[dsti.at[i]], add=True)
        return 0

    lax.fori_loop(0, NCHUNK, _chunk, 0)
    plsc.subcore_barrier()

    @pl.when(sid < NF)
    def _flush():
        pltpu.sync_copy(acc.at[pl.ds(sid * NPW, NPW)],
                        out_hbm.at[cid, pl.ds(sid * NPW, NPW)])


@functools.partial(
    pl.kernel,
    out_type=jax.ShapeDtypeStruct((NC, N, H), jnp.float32),
    mesh=_mesh,
    scratch_types=[
        pltpu.VMEM((NCHUNK, CH), jnp.int32),
        pltpu.VMEM((NCHUNK, CH), jnp.int32),
        pltpu.VMEM((KS, CH, H), jnp.float32),
        pltpu.VMEM((ZB, H), jnp.float32),
        pltpu.VMEM_SHARED((N, H), jnp.float32),
    ] + [pltpu.SemaphoreType.DMA] * (2 * KS),
    compiler_params=_sc_params,
)
def sc_scatter(h_hbm, src3_hbm, dst3_hbm, out_hbm,
               srci, dsti, rv, zbuf, acc, *sems):
    gsems = sems[:KS]
    ssems = sems[KS:]
    cid = lax.axis_index("c")
    sid = lax.axis_index("s")
    wid = sid * NC + cid

    zeros = jnp.zeros((16,), jnp.float32)

    pltpu.sync_copy(src3_hbm.at[wid], srci)
    pltpu.sync_copy(dst3_hbm.at[wid], dsti)

    @pl.when(sid < NF)
    def _init():
        def _zero(i, _):
            r = i // (H // 16)
            k = (i % (H // 16)) * 16
            zbuf[r, pl.ds(k, 16)] = zeros
            return 0

        lax.fori_loop(0, ZB * (H // 16), _zero, 0)
        for rr in range(NPW // ZB):
            pltpu.sync_copy(zbuf, acc.at[pl.ds(sid * NPW + rr * ZB, ZB)])

    plsc.subcore_barrier()

    def _round(t, _):
        gds = []
        for b in range(KS):
            i = t * KS + b
            d = pltpu.make_async_copy(h_hbm.at[srci.at[i]], rv.at[b],
                                      gsems[b])
            d.start()
            gds.append(d)

        sds = []
        for b in range(KS):
            i = t * KS + b
            gds[b].wait()
            sds.append(pltpu.async_copy(rv.at[b], acc.at[dsti.at[i]],
                                        ssems[b], add=True))

        for b in range(KS):
            sds[b].wait()

        return 0

    lax.fori_loop(0, NCHUNK // KS, _round, 0)
    plsc.subcore_barrier()

    @pl.when(sid < NF)
    def _flush():
        pltpu.sync_copy(acc.at[pl.ds(sid * NPW, NPW)],
                        out_hbm.at[cid, pl.ds(sid * NPW, NPW)])


@functools.partial(
    pl.kernel,
    out_type=jax.ShapeDtypeStruct((E, H), jnp.float32),
    mesh=_mesh,
    scratch_types=[
        pltpu.VMEM((NCHUNK, CH), jnp.int32),
        pltpu.VMEM((NCHUNK, CH), jnp.int32),
        pltpu.VMEM((KE, CH, H), jnp.float32),
        pltpu.VMEM((KE, CH, H), jnp.float32),
        pltpu.VMEM((KE, CH, H), jnp.float32),
    ] + [pltpu.SemaphoreType.DMA] * (2 * KE),
    compiler_params=_sc_params,
)
def sc_edgemlp(atab, btab, src3_hbm, dst3_hbm, c_hbm, out_hbm,
               srci, dsti, av, bv, cv, *sems):
    cid = lax.axis_index("c")
    sid = lax.axis_index("s")
    wid = sid * NC + cid

    gsems = sems[:KE]
    ssems = sems[KE:]

    pltpu.sync_copy(src3_hbm.at[wid], srci)
    pltpu.sync_copy(dst3_hbm.at[wid], dsti)

    def _round(t, _):
        ds = []
        for b in range(KE):
            i = t * KE + b
            base = wid * EPW + i * CH
            da = pltpu.make_async_copy(atab.at[srci.at[i]], av.at[b],
                                       gsems[b])
            db = pltpu.make_async_copy(btab.at[dsti.at[i]], bv.at[b],
                                       gsems[b])
            dc = pltpu.make_async_copy(c_hbm.at[pl.ds(base, CH)], cv.at[b],
                                       gsems[b])
            da.start()
            db.start()
            dc.start()
            ds.append((da, db, dc))

        sds = []
        for b in range(KE):
            i = t * KE + b
            base = wid * EPW + i * CH
            da, db, dc = ds[b]
            da.wait()
            db.wait()
            dc.wait()

            def _ew(r, _, b=b):
                for k in range(H // 16):
                    sl = pl.ds(k * 16, 16)
                    cv[b, r, sl] = jnp.maximum(
                        av[b, r, sl] + bv[b, r, sl] + cv[b, r, sl], 0.0)
                return 0

            lax.fori_loop(0, CH, _ew, 0, unroll=2)
            sds.append(pltpu.async_copy(cv.at[b],
                                        out_hbm.at[pl.ds(base, CH)],
                                        ssems[b]))

        for b in range(KE):
            sds[b].wait()

        return 0

    lax.fori_loop(0, NCHUNK // KE, _round, 0)


def _full(shape):
    return pl.BlockSpec(shape, lambda i: tuple(0 for _ in shape))


def tc_node_encoder(x, Wn1, bn1, Wn2, bn2):
    blk = 1000

    def body(x_ref, w1_ref, b1_ref, w2_ref, b2_ref, out_ref):
        h = jnp.maximum(x_ref[...] @ w1_ref[...] + b1_ref[...], 0.0)
        out_ref[...] = h @ w2_ref[...] + b2_ref[...]

    return pl.pallas_call(
        body,
        grid=(N // blk,),
        in_specs=[pl.BlockSpec((blk, 128), lambda i: (i, 0)),
                  _full((128, 32)), _full((1, 32)),
                  _full((32, H)), _full((1, H))],
        out_specs=pl.BlockSpec((blk, H), lambda i: (i, 0)),
        out_shape=jax.ShapeDtypeStruct((N, H), jnp.float32),
    )(x, Wn1, bn1, Wn2, bn2)


def tc_edge_encoder(e, We1, be1, We2, be2):
    blk = 1600

    def body(e_ref, w1_ref, b1_ref, w2_ref, b2_ref, out_ref):
        h = jnp.maximum(e_ref[...] @ w1_ref[...] + b1_ref[...], 0.0)
        out_ref[...] = h @ w2_ref[...] + b2_ref[...]

    return pl.pallas_call(
        body,
        grid=(E // blk,),
        in_specs=[pl.BlockSpec((blk, 16), lambda i: (i, 0)),
                  _full((16, 32)), _full((1, 32)),
                  _full((32, H)), _full((1, H))],
        out_specs=pl.BlockSpec((blk, H), lambda i: (i, 0)),
        out_shape=jax.ShapeDtypeStruct((E, H), jnp.float32),
    )(e, We1, be1, We2, be2)


def tc_mm64(xmat, W, b):
    blk = 1600

    def body(x_ref, w_ref, b_ref, out_ref):
        out_ref[...] = x_ref[...] @ w_ref[...] + b_ref[...]

    return pl.pallas_call(
        body,
        grid=(E // blk,),
        in_specs=[pl.BlockSpec((blk, H), lambda i: (i, 0)),
                  _full((H, H)), _full((1, H))],
        out_specs=pl.BlockSpec((blk, H), lambda i: (i, 0)),
        out_shape=jax.ShapeDtypeStruct((E, H), jnp.float32),
    )(xmat, W, b)


def tc_h_update(h, parts, degp, Wg_l, bg_l, W1, W2):
    blk = 1000

    def body(h_ref, p_ref, d_ref, wg_ref, bg_ref, w1_ref, w2_ref,
             hn_ref, a_ref, b_ref):
        h_ = h_ref[...]
        agg = p_ref[0] + p_ref[1]
        cnt = d_ref[0, :, :1] + d_ref[1, :, :1]
        inv = 1.0 / (cnt + 1.0)
        hn = jnp.maximum((h_ + (agg + h_) * inv) @ wg_ref[...] + bg_ref[...],
                         0.0)
        hn_ref[...] = hn
        a_ref[...] = hn @ w1_ref[...]
        b_ref[...] = hn @ w2_ref[...]

    return pl.pallas_call(
        body,
        grid=(N // blk,),
        in_specs=[pl.BlockSpec((blk, H), lambda i: (i, 0)),
                  pl.BlockSpec((NC, blk, H), lambda i: (0, i, 0)),
                  pl.BlockSpec((NC, blk, DW), lambda i: (0, i, 0)),
                  _full((H, H)), _full((1, H)), _full((H, H)), _full((H, H))],
        out_specs=[pl.BlockSpec((blk, H), lambda i: (i, 0)),
                   pl.BlockSpec((blk, H), lambda i: (i, 0)),
                   pl.BlockSpec((blk, H), lambda i: (i, 0))],
        out_shape=[jax.ShapeDtypeStruct((N, H), jnp.float32),
                   jax.ShapeDtypeStruct((N, H), jnp.float32),
                   jax.ShapeDtypeStruct((N, H), jnp.float32)],
    )(h, parts, degp, Wg_l, bg_l, W1, W2)


def tc_pair(h, P1, P2):
    blk = 1000

    def body(h_ref, w1_ref, w2_ref, a_ref, b_ref):
        h_ = h_ref[...]
        a_ref[...] = h_ @ w1_ref[...]
        b_ref[...] = h_ @ w2_ref[...]

    return pl.pallas_call(
        body,
        grid=(N // blk,),
        in_specs=[pl.BlockSpec((blk, H), lambda i: (i, 0)),
                  _full((H, H)), _full((H, H))],
        out_specs=[pl.BlockSpec((blk, H), lambda i: (i, 0)),
                   pl.BlockSpec((blk, H), lambda i: (i, 0))],
        out_shape=[jax.ShapeDtypeStruct((N, H), jnp.float32),
                   jax.ShapeDtypeStruct((N, H), jnp.float32)],
    )(h, P1, P2)


def tc_matvec(t, wrow, bp2):
    blk = 512

    def body(t_ref, w_ref, b_ref, out_ref):
        out_ref[...] = jnp.sum(t_ref[...] * w_ref[...], axis=1) + b_ref[0, 0]

    return pl.pallas_call(
        body,
        grid=(E // blk,),
        in_specs=[pl.BlockSpec((blk, H), lambda i: (i, 0)),
                  _full((1, H)), _full((1, 1))],
        out_specs=pl.BlockSpec((blk,), lambda i: (i,)),
        out_shape=jax.ShapeDtypeStruct((E,), jnp.float32),
    )(t, wrow, bp2)


def kernel(x, e, edge_index, Wn1, bn1, Wn2, bn2, We1, be1, We2, be2,
           Wg, bg, Weg, beg, Wp1, bp1, Wp2, bp2):
    L = Wg.shape[0]
    src = edge_index[0]
    dst = edge_index[1]

    src3 = src.reshape(NW, NCHUNK, CH)
    dst3 = dst.reshape(NW, NCHUNK, CH)

    h = tc_node_encoder(x, Wn1, bn1.reshape(1, -1), Wn2, bn2.reshape(1, -1))
    eh = tc_edge_encoder(e, We1, be1.reshape(1, -1), We2, be2.reshape(1, -1))
    degp = sc_deg(dst3)

    for l in range(L):
        W1 = Weg[l][:H]
        W2 = Weg[l][H:2 * H]
        W3 = Weg[l][2 * H:]
        e3 = tc_mm64(eh, W3, beg[l].reshape(1, -1))
        parts = sc_scatter(h, src3, dst3)
        h, hs1, hs2 = tc_h_update(h, parts, degp, Wg[l], bg[l].reshape(1, -1),
                                  W1, W2)
        eh = sc_edgemlp(hs1, hs2, src3, dst3, e3)

    p3 = tc_mm64(eh, Wp1[2 * H:], bp1.reshape(1, -1))
    hp1, hp2 = tc_pair(h, Wp1[:H], Wp1[H:2 * H])
    t = sc_edgemlp(hp1, hp2, src3, dst3, p3)
    return tc_matvec(t, Wp2.reshape(1, -1), bp2.reshape(1, 1))


# speedup vs baseline: 1.1703x; 1.1703x over previous
import functools

import jax
import jax.numpy as jnp
from jax import lax
from jax.experimental import pallas as pl
from jax.experimental.pallas import tpu as pltpu
from jax.experimental.pallas import tpu_sc as plsc

N = 10000
E = 320000
H = 64
DW = 16
NC = 2
NS = 16
NW = NC * NS
EPW = E // NW
CH = 80
NCHUNK = EPW // CH
NF = 10
NPW = N // NF
KS = 5
KE = 5
GLAG = 2
ZB = 200

_mesh = plsc.VectorSubcoreMesh(
    core_axis_name="c", subcore_axis_name="s", num_cores=NC, num_subcores=NS)

_sc_params = pltpu.CompilerParams(use_tc_tiling_on_sc=False)


@functools.partial(
    pl.kernel,
    out_type=jax.ShapeDtypeStruct((NC, N, DW), jnp.float32),
    mesh=_mesh,
    scratch_types=[
        pltpu.VMEM((NCHUNK, CH), jnp.int32),
        pltpu.VMEM((CH, DW), jnp.float32),
        pltpu.VMEM((NPW, DW), jnp.float32),
        pltpu.VMEM_SHARED((N, DW), jnp.float32),
    ],
    compiler_params=_sc_params,
)
def sc_deg(dst3_hbm, out_hbm, dsti, ones, zbuf, acc):
    cid = lax.axis_index("c")
    sid = lax.axis_index("s")
    wid = sid * NC + cid

    zeros = jnp.zeros((16,), jnp.float32)
    one16 = jnp.ones((16,), jnp.float32)

    def _fill(i, _):
        r = i // (DW // 16)
        k = (i % (DW // 16)) * 16
        ones[r, pl.ds(k, 16)] = one16
        return 0

    lax.fori_loop(0, CH * (DW // 16), _fill, 0)
    pltpu.sync_copy(dst3_hbm.at[wid], dsti)

    @pl.when(sid < NF)
    def _init():
        def _zero(i, _):
            r = i // (DW // 16)
            k = (i % (DW // 16)) * 16
            zbuf[r, pl.ds(k, 16)] = zeros
            return 0

        lax.fori_loop(0, NPW * (DW // 16), _zero, 0)
        pltpu.sync_copy(zbuf, acc.at[pl.ds(sid * NPW, NPW)])

    plsc.subcore_barrier()

    def _chunk(i, _):
        pltpu.sync_copy(ones, acc.at[dsti.at[i]], add=True)
        return 0

    lax.fori_loop(0, NCHUNK, _chunk, 0)
    plsc.subcore_barrier()

    @pl.when(sid < NF)
    def _flush():
        pltpu.sync_copy(acc.at[pl.ds(sid * NPW, NPW)],
                        out_hbm.at[cid, pl.ds(sid * NPW, NPW)])


@functools.partial(
    pl.kernel,
    out_type=jax.ShapeDtypeStruct((NC, N, H), jnp.float32),
    mesh=_mesh,
    scratch_types=[
        pltpu.VMEM((NCHUNK, CH), jnp.int32),
        pltpu.VMEM((NCHUNK, CH), jnp.int32),
        pltpu.VMEM((KS, CH, H), jnp.float32),
        pltpu.VMEM((ZB, H), jnp.float32),
        pltpu.VMEM_SHARED((N, H), jnp.float32),
    ] + [pltpu.SemaphoreType.DMA] * (2 * KS),
    compiler_params=_sc_params,
)
def sc_scatter(h_hbm, src3_hbm, dst3_hbm, out_hbm,
               srci, dsti, rv, zbuf, acc, *sems):
    gsems = sems[:KS]
    ssems = sems[KS:]
    cid = lax.axis_index("c")
    sid = lax.axis_index("s")
    wid = sid * NC + cid

    zeros = jnp.zeros((16,), jnp.float32)

    pltpu.sync_copy(src3_hbm.at[wid], srci)
    pltpu.sync_copy(dst3_hbm.at[wid], dsti)

    @pl.when(sid < NF)
    def _init():
        def _zero(i, _):
            r = i // (H // 16)
            k = (i % (H // 16)) * 16
            zbuf[r, pl.ds(k, 16)] = zeros
            return 0

        lax.fori_loop(0, ZB * (H // 16), _zero, 0)
        for rr in range(NPW // ZB):
            pltpu.sync_copy(zbuf, acc.at[pl.ds(sid * NPW + rr * ZB, ZB)])

    plsc.subcore_barrier()

    def _round(t, _):
        gds = []
        for b in range(KS):
            i = t * KS + b
            d = pltpu.make_async_copy(h_hbm.at[srci.at[i]], rv.at[b],
                                      gsems[b])
            d.start()
            gds.append(d)

        sds = []
        for b in range(KS):
            i = t * KS + b
            gds[b].wait()
            sds.append(pltpu.async_copy(rv.at[b], acc.at[dsti.at[i]],
                                        ssems[b], add=True))

        for b in range(KS):
            sds[b].wait()

        return 0

    lax.fori_loop(0, NCHUNK // KS, _round, 0)
    plsc.subcore_barrier()

    @pl.when(sid < NF)
    def _flush():
        pltpu.sync_copy(acc.at[pl.ds(sid * NPW, NPW)],
                        out_hbm.at[cid, pl.ds(sid * NPW, NPW)])


@functools.partial(
    pl.kernel,
    out_type=jax.ShapeDtypeStruct((E, H), jnp.float32),
    mesh=_mesh,
    scratch_types=[
        pltpu.VMEM((NCHUNK, CH), jnp.int32),
        pltpu.VMEM((NCHUNK, CH), jnp.int32),
        pltpu.VMEM((KE, CH, H), jnp.float32),
        pltpu.VMEM((KE, CH, H), jnp.float32),
        pltpu.VMEM((KE, CH, H), jnp.float32),
    ] + [pltpu.SemaphoreType.DMA] * (2 * KE),
    compiler_params=_sc_params,
)
def sc_edgemlp(atab, btab, src3_hbm, dst3_hbm, c_hbm, out_hbm,
               srci, dsti, av, bv, cv, *sems):
    cid = lax.axis_index("c")
    sid = lax.axis_index("s")
    wid = sid * NC + cid

    gsems = sems[:KE]
    ssems = sems[KE:]

    pltpu.sync_copy(src3_hbm.at[wid], srci)
    pltpu.sync_copy(dst3_hbm.at[wid], dsti)

    def _round(t, _):
        for b in range(KE):
            i = t * KE + b

            @pl.when(i >= KE)
            def _drain(i=i, b=b):
                base = wid * EPW + (i - KE) * CH
                pltpu.make_async_copy(
                    cv.at[b], out_hbm.at[pl.ds(base, CH)], ssems[b]).wait()

            @pl.when(i < NCHUNK)
            def _issue(i=i, b=b):
                base = wid * EPW + i * CH
                pltpu.async_copy(atab.at[srci.at[i]], av.at[b], gsems[b])
                pltpu.async_copy(btab.at[dsti.at[i]], bv.at[b], gsems[b])
                pltpu.async_copy(c_hbm.at[pl.ds(base, CH)], cv.at[b],
                                 gsems[b])

            c = i - GLAG
            q = (b - GLAG) % KE

            @pl.when((c >= 0) & (c < NCHUNK))
            def _compute(c=c, q=q):
                base = wid * EPW + c * CH
                pltpu.make_async_copy(
                    atab.at[srci.at[c]], av.at[q], gsems[q]).wait()
                pltpu.make_async_copy(
                    btab.at[dsti.at[c]], bv.at[q], gsems[q]).wait()
                pltpu.make_async_copy(
                    c_hbm.at[pl.ds(base, CH)], cv.at[q], gsems[q]).wait()

                def _ew(r, _, q=q):
                    for k in range(H // 16):
                        sl = pl.ds(k * 16, 16)
                        cv[q, r, sl] = jnp.maximum(
                            av[q, r, sl] + bv[q, r, sl] + cv[q, r, sl], 0.0)
                    return 0

                lax.fori_loop(0, CH, _ew, 0, unroll=2)
                pltpu.async_copy(cv.at[q], out_hbm.at[pl.ds(base, CH)],
                                 ssems[q])

        return 0

    lax.fori_loop(0, (NCHUNK + KE + KE - 1) // KE, _round, 0)


def _full(shape):
    return pl.BlockSpec(shape, lambda i: tuple(0 for _ in shape))


def tc_node_encoder(x, Wn1, bn1, Wn2, bn2):
    blk = 1000

    def body(x_ref, w1_ref, b1_ref, w2_ref, b2_ref, out_ref):
        h = jnp.maximum(x_ref[...] @ w1_ref[...] + b1_ref[...], 0.0)
        out_ref[...] = h @ w2_ref[...] + b2_ref[...]

    return pl.pallas_call(
        body,
        grid=(N // blk,),
        in_specs=[pl.BlockSpec((blk, 128), lambda i: (i, 0)),
                  _full((128, 32)), _full((1, 32)),
                  _full((32, H)), _full((1, H))],
        out_specs=pl.BlockSpec((blk, H), lambda i: (i, 0)),
        out_shape=jax.ShapeDtypeStruct((N, H), jnp.float32),
    )(x, Wn1, bn1, Wn2, bn2)


def tc_edge_encoder(e, We1, be1, We2, be2):
    blk = 1600

    def body(e_ref, w1_ref, b1_ref, w2_ref, b2_ref, out_ref):
        h = jnp.maximum(e_ref[...] @ w1_ref[...] + b1_ref[...], 0.0)
        out_ref[...] = h @ w2_ref[...] + b2_ref[...]

    return pl.pallas_call(
        body,
        grid=(E // blk,),
        in_specs=[pl.BlockSpec((blk, 16), lambda i: (i, 0)),
                  _full((16, 32)), _full((1, 32)),
                  _full((32, H)), _full((1, H))],
        out_specs=pl.BlockSpec((blk, H), lambda i: (i, 0)),
        out_shape=jax.ShapeDtypeStruct((E, H), jnp.float32),
    )(e, We1, be1, We2, be2)


def tc_mm64(xmat, W, b):
    blk = 1600

    def body(x_ref, w_ref, b_ref, out_ref):
        out_ref[...] = x_ref[...] @ w_ref[...] + b_ref[...]

    return pl.pallas_call(
        body,
        grid=(E // blk,),
        in_specs=[pl.BlockSpec((blk, H), lambda i: (i, 0)),
                  _full((H, H)), _full((1, H))],
        out_specs=pl.BlockSpec((blk, H), lambda i: (i, 0)),
        out_shape=jax.ShapeDtypeStruct((E, H), jnp.float32),
    )(xmat, W, b)


def tc_h_update(h, parts, degp, Wg_l, bg_l, W1, W2):
    blk = 1000

    def body(h_ref, p_ref, d_ref, wg_ref, bg_ref, w1_ref, w2_ref,
             hn_ref, a_ref, b_ref):
        h_ = h_ref[...]
        agg = p_ref[0] + p_ref[1]
        cnt = d_ref[0, :, :1] + d_ref[1, :, :1]
        inv = 1.0 / (cnt + 1.0)
        hn = jnp.maximum((h_ + (agg + h_) * inv) @ wg_ref[...] + bg_ref[...],
                         0.0)
        hn_ref[...] = hn
        a_ref[...] = hn @ w1_ref[...]
        b_ref[...] = hn @ w2_ref[...]

    return pl.pallas_call(
        body,
        grid=(N // blk,),
        in_specs=[pl.BlockSpec((blk, H), lambda i: (i, 0)),
                  pl.BlockSpec((NC, blk, H), lambda i: (0, i, 0)),
                  pl.BlockSpec((NC, blk, DW), lambda i: (0, i, 0)),
                  _full((H, H)), _full((1, H)), _full((H, H)), _full((H, H))],
        out_specs=[pl.BlockSpec((blk, H), lambda i: (i, 0)),
                   pl.BlockSpec((blk, H), lambda i: (i, 0)),
                   pl.BlockSpec((blk, H), lambda i: (i, 0))],
        out_shape=[jax.ShapeDtypeStruct((N, H), jnp.float32),
                   jax.ShapeDtypeStruct((N, H), jnp.float32),
                   jax.ShapeDtypeStruct((N, H), jnp.float32)],
    )(h, parts, degp, Wg_l, bg_l, W1, W2)


def tc_pair(h, P1, P2):
    blk = 1000

    def body(h_ref, w1_ref, w2_ref, a_ref, b_ref):
        h_ = h_ref[...]
        a_ref[...] = h_ @ w1_ref[...]
        b_ref[...] = h_ @ w2_ref[...]

    return pl.pallas_call(
        body,
        grid=(N // blk,),
        in_specs=[pl.BlockSpec((blk, H), lambda i: (i, 0)),
                  _full((H, H)), _full((H, H))],
        out_specs=[pl.BlockSpec((blk, H), lambda i: (i, 0)),
                   pl.BlockSpec((blk, H), lambda i: (i, 0))],
        out_shape=[jax.ShapeDtypeStruct((N, H), jnp.float32),
                   jax.ShapeDtypeStruct((N, H), jnp.float32)],
    )(h, P1, P2)


def tc_matvec(t, wrow, bp2):
    blk = 512

    def body(t_ref, w_ref, b_ref, out_ref):
        out_ref[...] = jnp.sum(t_ref[...] * w_ref[...], axis=1) + b_ref[0, 0]

    return pl.pallas_call(
        body,
        grid=(E // blk,),
        in_specs=[pl.BlockSpec((blk, H), lambda i: (i, 0)),
                  _full((1, H)), _full((1, 1))],
        out_specs=pl.BlockSpec((blk,), lambda i: (i,)),
        out_shape=jax.ShapeDtypeStruct((E,), jnp.float32),
    )(t, wrow, bp2)


def kernel(x, e, edge_index, Wn1, bn1, Wn2, bn2, We1, be1, We2, be2,
           Wg, bg, Weg, beg, Wp1, bp1, Wp2, bp2):
    L = Wg.shape[0]
    src = edge_index[0]
    dst = edge_index[1]

    src3 = src.reshape(NW, NCHUNK, CH)
    dst3 = dst.reshape(NW, NCHUNK, CH)

    h = tc_node_encoder(x, Wn1, bn1.reshape(1, -1), Wn2, bn2.reshape(1, -1))
    eh = tc_edge_encoder(e, We1, be1.reshape(1, -1), We2, be2.reshape(1, -1))
    degp = sc_deg(dst3)

    for l in range(L):
        W1 = Weg[l][:H]
        W2 = Weg[l][H:2 * H]
        W3 = Weg[l][2 * H:]
        e3 = tc_mm64(eh, W3, beg[l].reshape(1, -1))
        parts = sc_scatter(h, src3, dst3)
        h, hs1, hs2 = tc_h_update(h, parts, degp, Wg[l], bg[l].reshape(1, -1),
                                  W1, W2)
        eh = sc_edgemlp(hs1, hs2, src3, dst3, e3)

    p3 = tc_mm64(eh, Wp1[2 * H:], bp1.reshape(1, -1))
    hp1, hp2 = tc_pair(h, Wp1[:H], Wp1[H:2 * H])
    t = sc_edgemlp(hp1, hp2, src3, dst3, p3)
    return tc_matvec(t, Wp2.reshape(1, -1), bp2.reshape(1, 1))


# speedup vs baseline: 1.1754x; 1.0043x over previous
import functools

import jax
import jax.numpy as jnp
from jax import lax
from jax.experimental import pallas as pl
from jax.experimental.pallas import tpu as pltpu
from jax.experimental.pallas import tpu_sc as plsc

N = 10000
E = 320000
H = 64
DW = 16
NC = 2
NS = 16
NW = NC * NS
EPW = E // NW
CH = 80
NCHUNK = EPW // CH
NF = 10
NPW = N // NF
KS = 5
KE = 5
GLAG = 2
ZB = 200

_mesh = plsc.VectorSubcoreMesh(
    core_axis_name="c", subcore_axis_name="s", num_cores=NC, num_subcores=NS)

_sc_params = pltpu.CompilerParams(use_tc_tiling_on_sc=False)


@functools.partial(
    pl.kernel,
    out_type=jax.ShapeDtypeStruct((NC, N, DW), jnp.float32),
    mesh=_mesh,
    scratch_types=[
        pltpu.VMEM((NCHUNK, CH), jnp.int32),
        pltpu.VMEM((CH, DW), jnp.float32),
        pltpu.VMEM((NPW, DW), jnp.float32),
        pltpu.VMEM_SHARED((N, DW), jnp.float32),
        pltpu.SemaphoreType.DMA,
    ],
    compiler_params=_sc_params,
)
def sc_deg(dst3_hbm, out_hbm, dsti, ones, zbuf, acc, dsem):
    cid = lax.axis_index("c")
    sid = lax.axis_index("s")
    wid = sid * NC + cid

    zeros = jnp.zeros((16,), jnp.float32)
    one16 = jnp.ones((16,), jnp.float32)

    def _fill(i, _):
        r = i // (DW // 16)
        k = (i % (DW // 16)) * 16
        ones[r, pl.ds(k, 16)] = one16
        return 0

    lax.fori_loop(0, CH * (DW // 16), _fill, 0)
    pltpu.sync_copy(dst3_hbm.at[wid], dsti)

    @pl.when(sid < NF)
    def _init():
        def _zero(i, _):
            r = i // (DW // 16)
            k = (i % (DW // 16)) * 16
            zbuf[r, pl.ds(k, 16)] = zeros
            return 0

        lax.fori_loop(0, NPW * (DW // 16), _zero, 0)
        pltpu.sync_copy(zbuf, acc.at[pl.ds(sid * NPW, NPW)])

    plsc.subcore_barrier()

    def _chunk(i, _):
        @pl.when(i >= 1)
        def _wprev(i=i):
            pltpu.make_async_copy(ones, acc.at[dsti.at[i - 1]], dsem).wait()

        pltpu.async_copy(ones, acc.at[dsti.at[i]], dsem, add=True)
        return 0

    lax.fori_loop(0, NCHUNK, _chunk, 0)
    pltpu.make_async_copy(ones, acc.at[dsti.at[NCHUNK - 1]], dsem).wait()
    plsc.subcore_barrier()

    @pl.when(sid < NF)
    def _flush():
        pltpu.sync_copy(acc.at[pl.ds(sid * NPW, NPW)],
                        out_hbm.at[cid, pl.ds(sid * NPW, NPW)])


@functools.partial(
    pl.kernel,
    out_type=jax.ShapeDtypeStruct((NC, N, H), jnp.float32),
    mesh=_mesh,
    scratch_types=[
        pltpu.VMEM((NCHUNK, CH), jnp.int32),
        pltpu.VMEM((NCHUNK, CH), jnp.int32),
        pltpu.VMEM((KS, CH, H), jnp.float32),
        pltpu.VMEM((ZB, H), jnp.float32),
        pltpu.VMEM_SHARED((N, H), jnp.float32),
    ] + [pltpu.SemaphoreType.DMA] * (2 * KS),
    compiler_params=_sc_params,
)
def sc_scatter(h_hbm, src3_hbm, dst3_hbm, out_hbm,
               srci, dsti, rv, zbuf, acc, *sems):
    gsems = sems[:KS]
    ssems = sems[KS:]
    cid = lax.axis_index("c")
    sid = lax.axis_index("s")
    wid = sid * NC + cid

    zeros = jnp.zeros((16,), jnp.float32)

    pltpu.sync_copy(src3_hbm.at[wid], srci)
    pltpu.sync_copy(dst3_hbm.at[wid], dsti)

    @pl.when(sid < NF)
    def _init():
        def _zero(i, _):
            r = i // (H // 16)
            k = (i % (H // 16)) * 16
            zbuf[r, pl.ds(k, 16)] = zeros
            return 0

        lax.fori_loop(0, ZB * (H // 16), _zero, 0)
        for rr in range(NPW // ZB):
            pltpu.sync_copy(zbuf, acc.at[pl.ds(sid * NPW + rr * ZB, ZB)])

    plsc.subcore_barrier()

    def _round(t, _):
        for b in range(KS):
            i = t * KS + b

            @pl.when(i < NCHUNK)
            def _issue(i=i, b=b):
                pltpu.async_copy(h_hbm.at[srci.at[i]], rv.at[b], gsems[b])

            c = i - GLAG
            q = (b - GLAG) % KS

            @pl.when((c >= 0) & (c < NCHUNK))
            def _scat(c=c, q=q):
                pltpu.make_async_copy(
                    h_hbm.at[srci.at[c]], rv.at[q], gsems[q]).wait()

                @pl.when(c >= 1)
                def _wprev(c=c, q=q):
                    qp = (q - 1) % KS
                    pltpu.make_async_copy(
                        rv.at[qp], acc.at[dsti.at[c - 1]], ssems[0]).wait()

                pltpu.async_copy(rv.at[q], acc.at[dsti.at[c]], ssems[0],
                                 add=True)

        return 0

    lax.fori_loop(0, (NCHUNK + GLAG + KS - 1) // KS, _round, 0)
    pltpu.make_async_copy(rv.at[(NCHUNK - 1) % KS],
                          acc.at[dsti.at[NCHUNK - 1]], ssems[0]).wait()
    plsc.subcore_barrier()

    @pl.when(sid < NF)
    def _flush():
        pltpu.sync_copy(acc.at[pl.ds(sid * NPW, NPW)],
                        out_hbm.at[cid, pl.ds(sid * NPW, NPW)])


@functools.partial(
    pl.kernel,
    out_type=jax.ShapeDtypeStruct((E, H), jnp.float32),
    mesh=_mesh,
    scratch_types=[
        pltpu.VMEM((NCHUNK, CH), jnp.int32),
        pltpu.VMEM((NCHUNK, CH), jnp.int32),
        pltpu.VMEM((KE, CH, H), jnp.float32),
        pltpu.VMEM((KE, CH, H), jnp.float32),
        pltpu.VMEM((KE, CH, H), jnp.float32),
    ] + [pltpu.SemaphoreType.DMA] * (2 * KE),
    compiler_params=_sc_params,
)
def sc_edgemlp(atab, btab, src3_hbm, dst3_hbm, c_hbm, out_hbm,
               srci, dsti, av, bv, cv, *sems):
    cid = lax.axis_index("c")
    sid = lax.axis_index("s")
    wid = sid * NC + cid

    gsems = sems[:KE]
    ssems = sems[KE:]

    pltpu.sync_copy(src3_hbm.at[wid], srci)
    pltpu.sync_copy(dst3_hbm.at[wid], dsti)

    def _round(t, _):
        for b in range(KE):
            i = t * KE + b

            @pl.when(i >= KE)
            def _drain(i=i, b=b):
                base = wid * EPW + (i - KE) * CH
                pltpu.make_async_copy(
                    cv.at[b], out_hbm.at[pl.ds(base, CH)], ssems[b]).wait()

            @pl.when(i < NCHUNK)
            def _issue(i=i, b=b):
                base = wid * EPW + i * CH
                pltpu.async_copy(atab.at[srci.at[i]], av.at[b], gsems[b])
                pltpu.async_copy(btab.at[dsti.at[i]], bv.at[b], gsems[b])
                pltpu.async_copy(c_hbm.at[pl.ds(base, CH)], cv.at[b],
                                 gsems[b])

            c = i - GLAG
            q = (b - GLAG) % KE

            @pl.when((c >= 0) & (c < NCHUNK))
            def _compute(c=c, q=q):
                base = wid * EPW + c * CH
                pltpu.make_async_copy(
                    atab.at[srci.at[c]], av.at[q], gsems[q]).wait()
                pltpu.make_async_copy(
                    btab.at[dsti.at[c]], bv.at[q], gsems[q]).wait()
                pltpu.make_async_copy(
                    c_hbm.at[pl.ds(base, CH)], cv.at[q], gsems[q]).wait()

                def _ew(r, _, q=q):
                    for k in range(H // 16):
                        sl = pl.ds(k * 16, 16)
                        cv[q, r, sl] = jnp.maximum(
                            av[q, r, sl] + bv[q, r, sl] + cv[q, r, sl], 0.0)
                    return 0

                lax.fori_loop(0, CH, _ew, 0, unroll=2)
                pltpu.async_copy(cv.at[q], out_hbm.at[pl.ds(base, CH)],
                                 ssems[q])

        return 0

    lax.fori_loop(0, (NCHUNK + KE + KE - 1) // KE, _round, 0)


def _full(shape):
    return pl.BlockSpec(shape, lambda i: tuple(0 for _ in shape))


def tc_node_encoder(x, Wn1, bn1, Wn2, bn2):
    blk = 1000

    def body(x_ref, w1_ref, b1_ref, w2_ref, b2_ref, out_ref):
        h = jnp.maximum(x_ref[...] @ w1_ref[...] + b1_ref[...], 0.0)
        out_ref[...] = h @ w2_ref[...] + b2_ref[...]

    return pl.pallas_call(
        body,
        grid=(N // blk,),
        in_specs=[pl.BlockSpec((blk, 128), lambda i: (i, 0)),
                  _full((128, 32)), _full((1, 32)),
                  _full((32, H)), _full((1, H))],
        out_specs=pl.BlockSpec((blk, H), lambda i: (i, 0)),
        out_shape=jax.ShapeDtypeStruct((N, H), jnp.float32),
    )(x, Wn1, bn1, Wn2, bn2)


def tc_edge_encoder(e, We1, be1, We2, be2):
    blk = 1600

    def body(e_ref, w1_ref, b1_ref, w2_ref, b2_ref, out_ref):
        h = jnp.maximum(e_ref[...] @ w1_ref[...] + b1_ref[...], 0.0)
        out_ref[...] = h @ w2_ref[...] + b2_ref[...]

    return pl.pallas_call(
        body,
        grid=(E // blk,),
        in_specs=[pl.BlockSpec((blk, 16), lambda i: (i, 0)),
                  _full((16, 32)), _full((1, 32)),
                  _full((32, H)), _full((1, H))],
        out_specs=pl.BlockSpec((blk, H), lambda i: (i, 0)),
        out_shape=jax.ShapeDtypeStruct((E, H), jnp.float32),
    )(e, We1, be1, We2, be2)


def tc_mm64(xmat, W, b):
    blk = 1600

    def body(x_ref, w_ref, b_ref, out_ref):
        out_ref[...] = x_ref[...] @ w_ref[...] + b_ref[...]

    return pl.pallas_call(
        body,
        grid=(E // blk,),
        in_specs=[pl.BlockSpec((blk, H), lambda i: (i, 0)),
                  _full((H, H)), _full((1, H))],
        out_specs=pl.BlockSpec((blk, H), lambda i: (i, 0)),
        out_shape=jax.ShapeDtypeStruct((E, H), jnp.float32),
    )(xmat, W, b)


def tc_h_update(h, parts, degp, Wg_l, bg_l, W1, W2):
    blk = 1000

    def body(h_ref, p_ref, d_ref, wg_ref, bg_ref, w1_ref, w2_ref,
             hn_ref, a_ref, b_ref):
        h_ = h_ref[...]
        agg = p_ref[0] + p_ref[1]
        cnt = d_ref[0, :, :1] + d_ref[1, :, :1]
        inv = 1.0 / (cnt + 1.0)
        hn = jnp.maximum((h_ + (agg + h_) * inv) @ wg_ref[...] + bg_ref[...],
                         0.0)
        hn_ref[...] = hn
        a_ref[...] = hn @ w1_ref[...]
        b_ref[...] = hn @ w2_ref[...]

    return pl.pallas_call(
        body,
        grid=(N // blk,),
        in_specs=[pl.BlockSpec((blk, H), lambda i: (i, 0)),
                  pl.BlockSpec((NC, blk, H), lambda i: (0, i, 0)),
                  pl.BlockSpec((NC, blk, DW), lambda i: (0, i, 0)),
                  _full((H, H)), _full((1, H)), _full((H, H)), _full((H, H))],
        out_specs=[pl.BlockSpec((blk, H), lambda i: (i, 0)),
                   pl.BlockSpec((blk, H), lambda i: (i, 0)),
                   pl.BlockSpec((blk, H), lambda i: (i, 0))],
        out_shape=[jax.ShapeDtypeStruct((N, H), jnp.float32),
                   jax.ShapeDtypeStruct((N, H), jnp.float32),
                   jax.ShapeDtypeStruct((N, H), jnp.float32)],
    )(h, parts, degp, Wg_l, bg_l, W1, W2)


def tc_pair(h, P1, P2):
    blk = 1000

    def body(h_ref, w1_ref, w2_ref, a_ref, b_ref):
        h_ = h_ref[...]
        a_ref[...] = h_ @ w1_ref[...]
        b_ref[...] = h_ @ w2_ref[...]

    return pl.pallas_call(
        body,
        grid=(N // blk,),
        in_specs=[pl.BlockSpec((blk, H), lambda i: (i, 0)),
                  _full((H, H)), _full((H, H))],
        out_specs=[pl.BlockSpec((blk, H), lambda i: (i, 0)),
                   pl.BlockSpec((blk, H), lambda i: (i, 0))],
        out_shape=[jax.ShapeDtypeStruct((N, H), jnp.float32),
                   jax.ShapeDtypeStruct((N, H), jnp.float32)],
    )(h, P1, P2)


def tc_matvec(t, wrow, bp2):
    blk = 512

    def body(t_ref, w_ref, b_ref, out_ref):
        out_ref[...] = jnp.sum(t_ref[...] * w_ref[...], axis=1) + b_ref[0, 0]

    return pl.pallas_call(
        body,
        grid=(E // blk,),
        in_specs=[pl.BlockSpec((blk, H), lambda i: (i, 0)),
                  _full((1, H)), _full((1, 1))],
        out_specs=pl.BlockSpec((blk,), lambda i: (i,)),
        out_shape=jax.ShapeDtypeStruct((E,), jnp.float32),
    )(t, wrow, bp2)


def kernel(x, e, edge_index, Wn1, bn1, Wn2, bn2, We1, be1, We2, be2,
           Wg, bg, Weg, beg, Wp1, bp1, Wp2, bp2):
    L = Wg.shape[0]
    src = edge_index[0]
    dst = edge_index[1]

    src3 = src.reshape(NW, NCHUNK, CH)
    dst3 = dst.reshape(NW, NCHUNK, CH)

    h = tc_node_encoder(x, Wn1, bn1.reshape(1, -1), Wn2, bn2.reshape(1, -1))
    eh = tc_edge_encoder(e, We1, be1.reshape(1, -1), We2, be2.reshape(1, -1))
    degp = sc_deg(dst3)

    for l in range(L):
        W1 = Weg[l][:H]
        W2 = Weg[l][H:2 * H]
        W3 = Weg[l][2 * H:]
        e3 = tc_mm64(eh, W3, beg[l].reshape(1, -1))
        parts = sc_scatter(h, src3, dst3)
        h, hs1, hs2 = tc_h_update(h, parts, degp, Wg[l], bg[l].reshape(1, -1),
                                  W1, W2)
        eh = sc_edgemlp(hs1, hs2, src3, dst3, e3)

    p3 = tc_mm64(eh, Wp1[2 * H:], bp1.reshape(1, -1))
    hp1, hp2 = tc_pair(h, Wp1[:H], Wp1[H:2 * H])
    t = sc_edgemlp(hp1, hp2, src3, dst3, p3)
    return tc_matvec(t, Wp2.reshape(1, -1), bp2.reshape(1, 1))


# speedup vs baseline: 1.7187x; 1.4622x over previous
import functools

import jax
import jax.numpy as jnp
from jax import lax
from jax.experimental import pallas as pl
from jax.experimental.pallas import tpu as pltpu
from jax.experimental.pallas import tpu_sc as plsc

N = 10000
E = 320000
H = 64
DW = 16
NC = 2
NS = 16
NW = NC * NS
EPW = E // NW
CH = 80
NCHUNK = EPW // CH
NF = 10
NPW = N // NF
KS = 5
KE = 5
GLAG = 2
ZB = 200

_mesh = plsc.VectorSubcoreMesh(
    core_axis_name="c", subcore_axis_name="s", num_cores=NC, num_subcores=NS)

_sc_params = pltpu.CompilerParams(use_tc_tiling_on_sc=False)


@functools.partial(
    pl.kernel,
    out_type=jax.ShapeDtypeStruct((NC, N, DW), jnp.float32),
    mesh=_mesh,
    scratch_types=[
        pltpu.VMEM((NCHUNK, CH), jnp.int32),
        pltpu.VMEM((CH, DW), jnp.float32),
        pltpu.VMEM((NPW, DW), jnp.float32),
        pltpu.VMEM_SHARED((N, DW), jnp.float32),
        pltpu.SemaphoreType.DMA,
    ],
    compiler_params=_sc_params,
)
def sc_deg(dst3_hbm, out_hbm, dsti, ones, zbuf, acc, dsem):
    cid = lax.axis_index("c")
    sid = lax.axis_index("s")
    wid = sid * NC + cid

    zeros = jnp.zeros((16,), jnp.float32)
    one16 = jnp.ones((16,), jnp.float32)

    def _fill(i, _):
        r = i // (DW // 16)
        k = (i % (DW // 16)) * 16
        ones[r, pl.ds(k, 16)] = one16
        return 0

    lax.fori_loop(0, CH * (DW // 16), _fill, 0)
    pltpu.sync_copy(dst3_hbm.at[wid], dsti)

    @pl.when(sid < NF)
    def _init():
        def _zero(i, _):
            r = i // (DW // 16)
            k = (i % (DW // 16)) * 16
            zbuf[r, pl.ds(k, 16)] = zeros
            return 0

        lax.fori_loop(0, NPW * (DW // 16), _zero, 0)
        pltpu.sync_copy(zbuf, acc.at[pl.ds(sid * NPW, NPW)])

    plsc.subcore_barrier()

    def _chunk(i, _):
        @pl.when(i >= 1)
        def _wprev(i=i):
            pltpu.make_async_copy(ones, acc.at[dsti.at[i - 1]], dsem).wait()

        pltpu.async_copy(ones, acc.at[dsti.at[i]], dsem, add=True)
        return 0

    lax.fori_loop(0, NCHUNK, _chunk, 0)
    pltpu.make_async_copy(ones, acc.at[dsti.at[NCHUNK - 1]], dsem).wait()
    plsc.subcore_barrier()

    @pl.when(sid < NF)
    def _flush():
        pltpu.sync_copy(acc.at[pl.ds(sid * NPW, NPW)],
                        out_hbm.at[cid, pl.ds(sid * NPW, NPW)])


@functools.partial(
    pl.kernel,
    out_type=jax.ShapeDtypeStruct((NC, N, H), jnp.float32),
    mesh=_mesh,
    scratch_types=[
        pltpu.VMEM((NCHUNK, CH), jnp.int32),
        pltpu.VMEM((NCHUNK, CH), jnp.int32),
        pltpu.VMEM((KS, CH, H), jnp.float32),
        pltpu.VMEM((ZB, H), jnp.float32),
        pltpu.VMEM_SHARED((N, H), jnp.float32),
    ] + [pltpu.SemaphoreType.DMA] * (2 * KS),
    compiler_params=_sc_params,
)
def sc_scatter(h_hbm, src3_hbm, dst3_hbm, out_hbm,
               srci, dsti, rv, zbuf, acc, *sems):
    gsems = sems[:KS]
    ssems = sems[KS:]
    cid = lax.axis_index("c")
    sid = lax.axis_index("s")
    wid = sid * NC + cid

    zeros = jnp.zeros((16,), jnp.float32)

    pltpu.sync_copy(src3_hbm.at[wid], srci)
    pltpu.sync_copy(dst3_hbm.at[wid], dsti)

    @pl.when(sid < NF)
    def _init():
        def _zero(i, _):
            r = i // (H // 16)
            k = (i % (H // 16)) * 16
            zbuf[r, pl.ds(k, 16)] = zeros
            return 0

        lax.fori_loop(0, ZB * (H // 16), _zero, 0)
        for rr in range(NPW // ZB):
            pltpu.sync_copy(zbuf, acc.at[pl.ds(sid * NPW + rr * ZB, ZB)])

    plsc.subcore_barrier()

    def _round(t, _):
        for b in range(KS):
            i = t * KS + b

            @pl.when(i < NCHUNK)
            def _issue(i=i, b=b):
                pltpu.async_copy(h_hbm.at[srci.at[i]], rv.at[b], gsems[b])

            c = i - GLAG
            q = (b - GLAG) % KS

            @pl.when((c >= 0) & (c < NCHUNK))
            def _scat(c=c, q=q):
                pltpu.make_async_copy(
                    h_hbm.at[srci.at[c]], rv.at[q], gsems[q]).wait()

                @pl.when(c >= 1)
                def _wprev(c=c, q=q):
                    qp = (q - 1) % KS
                    pltpu.make_async_copy(
                        rv.at[qp], acc.at[dsti.at[c - 1]], ssems[0]).wait()

                pltpu.async_copy(rv.at[q], acc.at[dsti.at[c]], ssems[0],
                                 add=True)

        return 0

    lax.fori_loop(0, (NCHUNK + GLAG + KS - 1) // KS, _round, 0)
    pltpu.make_async_copy(rv.at[(NCHUNK - 1) % KS],
                          acc.at[dsti.at[NCHUNK - 1]], ssems[0]).wait()
    plsc.subcore_barrier()

    @pl.when(sid < NF)
    def _flush():
        pltpu.sync_copy(acc.at[pl.ds(sid * NPW, NPW)],
                        out_hbm.at[cid, pl.ds(sid * NPW, NPW)])


@functools.partial(
    pl.kernel,
    out_type=jax.ShapeDtypeStruct((E, H), jnp.float32),
    mesh=_mesh,
    scratch_types=[
        pltpu.VMEM((NCHUNK, CH), jnp.int32),
        pltpu.VMEM((NCHUNK, CH), jnp.int32),
        pltpu.VMEM((KE, CH, H), jnp.float32),
        pltpu.VMEM((KE, CH, H), jnp.float32),
        pltpu.VMEM((KE, CH, H), jnp.float32),
    ] + [pltpu.SemaphoreType.DMA] * (2 * KE),
    compiler_params=_sc_params,
)
def sc_gpair(atab, btab, src3_hbm, dst3_hbm, out_hbm,
             srci, dsti, av, bv, cv, *sems):
    cid = lax.axis_index("c")
    sid = lax.axis_index("s")
    wid = sid * NC + cid

    gsems = sems[:KE]
    ssems = sems[KE:]

    pltpu.sync_copy(src3_hbm.at[wid], srci)
    pltpu.sync_copy(dst3_hbm.at[wid], dsti)

    def _round(t, _):
        for b in range(KE):
            i = t * KE + b

            @pl.when(i >= KE)
            def _drain(i=i, b=b):
                base = wid * EPW + (i - KE) * CH
                pltpu.make_async_copy(
                    cv.at[b], out_hbm.at[pl.ds(base, CH)], ssems[b]).wait()

            @pl.when(i < NCHUNK)
            def _issue(i=i, b=b):
                pltpu.async_copy(atab.at[srci.at[i]], av.at[b], gsems[b])
                pltpu.async_copy(btab.at[dsti.at[i]], bv.at[b], gsems[b])

            c = i - GLAG
            q = (b - GLAG) % KE

            @pl.when((c >= 0) & (c < NCHUNK))
            def _compute(c=c, q=q):
                base = wid * EPW + c * CH
                pltpu.make_async_copy(
                    atab.at[srci.at[c]], av.at[q], gsems[q]).wait()
                pltpu.make_async_copy(
                    btab.at[dsti.at[c]], bv.at[q], gsems[q]).wait()

                def _ew(r, _, q=q):
                    for k in range(H // 16):
                        sl = pl.ds(k * 16, 16)
                        cv[q, r, sl] = av[q, r, sl] + bv[q, r, sl]
                    return 0

                lax.fori_loop(0, CH, _ew, 0, unroll=2)
                pltpu.async_copy(cv.at[q], out_hbm.at[pl.ds(base, CH)],
                                 ssems[q])

        return 0

    lax.fori_loop(0, (NCHUNK + KE + KE - 1) // KE, _round, 0)


def _full(shape):
    return pl.BlockSpec(shape, lambda i: tuple(0 for _ in shape))


def tc_node_encoder(x, Wn1, bn1, Wn2, bn2):
    blk = 1000

    def body(x_ref, w1_ref, b1_ref, w2_ref, b2_ref, out_ref):
        h = jnp.maximum(x_ref[...] @ w1_ref[...] + b1_ref[...], 0.0)
        out_ref[...] = h @ w2_ref[...] + b2_ref[...]

    return pl.pallas_call(
        body,
        grid=(N // blk,),
        in_specs=[pl.BlockSpec((blk, 128), lambda i: (i, 0)),
                  _full((128, 32)), _full((1, 32)),
                  _full((32, H)), _full((1, H))],
        out_specs=pl.BlockSpec((blk, H), lambda i: (i, 0)),
        out_shape=jax.ShapeDtypeStruct((N, H), jnp.float32),
    )(x, Wn1, bn1, Wn2, bn2)


def tc_edge_encoder(e, We1, be1, We2, be2):
    blk = 1600

    def body(e_ref, w1_ref, b1_ref, w2_ref, b2_ref, out_ref):
        h = jnp.maximum(e_ref[...] @ w1_ref[...] + b1_ref[...], 0.0)
        out_ref[...] = h @ w2_ref[...] + b2_ref[...]

    return pl.pallas_call(
        body,
        grid=(E // blk,),
        in_specs=[pl.BlockSpec((blk, 16), lambda i: (i, 0)),
                  _full((16, 32)), _full((1, 32)),
                  _full((32, H)), _full((1, H))],
        out_specs=pl.BlockSpec((blk, H), lambda i: (i, 0)),
        out_shape=jax.ShapeDtypeStruct((E, H), jnp.float32),
    )(e, We1, be1, We2, be2)


def tc_mlp_fused(g, eh, W, b):
    blk = 1600

    def body(g_ref, e_ref, w_ref, b_ref, out_ref):
        out_ref[...] = jnp.maximum(
            g_ref[...] + e_ref[...] @ w_ref[...] + b_ref[...], 0.0)

    return pl.pallas_call(
        body,
        grid=(E // blk,),
        in_specs=[pl.BlockSpec((blk, H), lambda i: (i, 0)),
                  pl.BlockSpec((blk, H), lambda i: (i, 0)),
                  _full((H, H)), _full((1, H))],
        out_specs=pl.BlockSpec((blk, H), lambda i: (i, 0)),
        out_shape=jax.ShapeDtypeStruct((E, H), jnp.float32),
    )(g, eh, W, b)


def tc_pred(g, eh, W, b1, wrow, bp2):
    blk = 512

    def body(g_ref, e_ref, w_ref, b1_ref, wr_ref, b2_ref, out_ref):
        t = jnp.maximum(g_ref[...] + e_ref[...] @ w_ref[...] + b1_ref[...],
                        0.0)
        out_ref[...] = jnp.sum(t * wr_ref[...], axis=1) + b2_ref[0, 0]

    return pl.pallas_call(
        body,
        grid=(E // blk,),
        in_specs=[pl.BlockSpec((blk, H), lambda i: (i, 0)),
                  pl.BlockSpec((blk, H), lambda i: (i, 0)),
                  _full((H, H)), _full((1, H)), _full((1, H)), _full((1, 1))],
        out_specs=pl.BlockSpec((blk,), lambda i: (i,)),
        out_shape=jax.ShapeDtypeStruct((E,), jnp.float32),
    )(g, eh, W, b1, wrow, bp2)


def tc_h_update(h, parts, degp, Wg_l, bg_l, W1, W2):
    blk = 1000

    def body(h_ref, p_ref, d_ref, wg_ref, bg_ref, w1_ref, w2_ref,
             hn_ref, a_ref, b_ref):
        h_ = h_ref[...]
        agg = p_ref[0] + p_ref[1]
        cnt = d_ref[0, :, :1] + d_ref[1, :, :1]
        inv = 1.0 / (cnt + 1.0)
        hn = jnp.maximum((h_ + (agg + h_) * inv) @ wg_ref[...] + bg_ref[...],
                         0.0)
        hn_ref[...] = hn
        a_ref[...] = hn @ w1_ref[...]
        b_ref[...] = hn @ w2_ref[...]

    return pl.pallas_call(
        body,
        grid=(N // blk,),
        in_specs=[pl.BlockSpec((blk, H), lambda i: (i, 0)),
                  pl.BlockSpec((NC, blk, H), lambda i: (0, i, 0)),
                  pl.BlockSpec((NC, blk, DW), lambda i: (0, i, 0)),
                  _full((H, H)), _full((1, H)), _full((H, H)), _full((H, H))],
        out_specs=[pl.BlockSpec((blk, H), lambda i: (i, 0)),
                   pl.BlockSpec((blk, H), lambda i: (i, 0)),
                   pl.BlockSpec((blk, H), lambda i: (i, 0))],
        out_shape=[jax.ShapeDtypeStruct((N, H), jnp.float32),
                   jax.ShapeDtypeStruct((N, H), jnp.float32),
                   jax.ShapeDtypeStruct((N, H), jnp.float32)],
    )(h, parts, degp, Wg_l, bg_l, W1, W2)


def tc_pair(h, P1, P2):
    blk = 1000

    def body(h_ref, w1_ref, w2_ref, a_ref, b_ref):
        h_ = h_ref[...]
        a_ref[...] = h_ @ w1_ref[...]
        b_ref[...] = h_ @ w2_ref[...]

    return pl.pallas_call(
        body,
        grid=(N // blk,),
        in_specs=[pl.BlockSpec((blk, H), lambda i: (i, 0)),
                  _full((H, H)), _full((H, H))],
        out_specs=[pl.BlockSpec((blk, H), lambda i: (i, 0)),
                   pl.BlockSpec((blk, H), lambda i: (i, 0))],
        out_shape=[jax.ShapeDtypeStruct((N, H), jnp.float32),
                   jax.ShapeDtypeStruct((N, H), jnp.float32)],
    )(h, P1, P2)


def kernel(x, e, edge_index, Wn1, bn1, Wn2, bn2, We1, be1, We2, be2,
           Wg, bg, Weg, beg, Wp1, bp1, Wp2, bp2):
    L = Wg.shape[0]
    src = edge_index[0]
    dst = edge_index[1]

    src3 = src.reshape(NW, NCHUNK, CH)
    dst3 = dst.reshape(NW, NCHUNK, CH)

    h = tc_node_encoder(x, Wn1, bn1.reshape(1, -1), Wn2, bn2.reshape(1, -1))
    eh = tc_edge_encoder(e, We1, be1.reshape(1, -1), We2, be2.reshape(1, -1))
    degp = sc_deg(dst3)

    for l in range(L):
        W1 = Weg[l][:H]
        W2 = Weg[l][H:2 * H]
        W3 = Weg[l][2 * H:]
        parts = sc_scatter(h, src3, dst3)
        h, hs1, hs2 = tc_h_update(h, parts, degp, Wg[l], bg[l].reshape(1, -1),
                                  W1, W2)
        g = sc_gpair(hs1, hs2, src3, dst3)
        eh = tc_mlp_fused(g, eh, W3, beg[l].reshape(1, -1))

    hp1, hp2 = tc_pair(h, Wp1[:H], Wp1[H:2 * H])
    gp = sc_gpair(hp1, hp2, src3, dst3)
    return tc_pred(gp, eh, Wp1[2 * H:], bp1.reshape(1, -1),
                   Wp2.reshape(1, -1), bp2.reshape(1, 1))


# speedup vs baseline: 1.7221x; 1.0020x over previous
import functools

import jax
import jax.numpy as jnp
from jax import lax
from jax.experimental import pallas as pl
from jax.experimental.pallas import tpu as pltpu
from jax.experimental.pallas import tpu_sc as plsc

N = 10000
E = 320000
H = 64
DW = 16
NC = 2
NS = 16
NW = NC * NS
EPW = E // NW
CH = 80
NCHUNK = EPW // CH
NF = 10
NPW = N // NF
CHD = 400
NCHD = EPW // CHD
KS = 5
KE = 5
GLAG = 3
ZB = 200

_mesh = plsc.VectorSubcoreMesh(
    core_axis_name="c", subcore_axis_name="s", num_cores=NC, num_subcores=NS)

_sc_params = pltpu.CompilerParams(use_tc_tiling_on_sc=False)


@functools.partial(
    pl.kernel,
    out_type=jax.ShapeDtypeStruct((NC, N, DW), jnp.float32),
    mesh=_mesh,
    scratch_types=[
        pltpu.VMEM((NCHD, CHD), jnp.int32),
        pltpu.VMEM((CHD, DW), jnp.float32),
        pltpu.VMEM((NPW, DW), jnp.float32),
        pltpu.VMEM_SHARED((N, DW), jnp.float32),
        pltpu.SemaphoreType.DMA,
    ],
    compiler_params=_sc_params,
)
def sc_deg(dst4_hbm, out_hbm, dsti, ones, zbuf, acc, dsem):
    cid = lax.axis_index("c")
    sid = lax.axis_index("s")
    wid = sid * NC + cid

    zeros = jnp.zeros((16,), jnp.float32)
    one16 = jnp.ones((16,), jnp.float32)

    def _fill(i, _):
        r = i // (DW // 16)
        k = (i % (DW // 16)) * 16
        ones[r, pl.ds(k, 16)] = one16
        return 0

    lax.fori_loop(0, CHD * (DW // 16), _fill, 0)
    pltpu.sync_copy(dst4_hbm.at[wid], dsti)

    @pl.when(sid < NF)
    def _init():
        def _zero(i, _):
            r = i // (DW // 16)
            k = (i % (DW // 16)) * 16
            zbuf[r, pl.ds(k, 16)] = zeros
            return 0

        lax.fori_loop(0, NPW * (DW // 16), _zero, 0)
        pltpu.sync_copy(zbuf, acc.at[pl.ds(sid * NPW, NPW)])

    plsc.subcore_barrier()

    def _chunk(i, _):
        @pl.when(i >= 1)
        def _wprev(i=i):
            pltpu.make_async_copy(ones, acc.at[dsti.at[i - 1]], dsem).wait()

        pltpu.async_copy(ones, acc.at[dsti.at[i]], dsem, add=True)
        return 0

    lax.fori_loop(0, NCHD, _chunk, 0)
    pltpu.make_async_copy(ones, acc.at[dsti.at[NCHD - 1]], dsem).wait()
    plsc.subcore_barrier()

    @pl.when(sid < NF)
    def _flush():
        pltpu.sync_copy(acc.at[pl.ds(sid * NPW, NPW)],
                        out_hbm.at[cid, pl.ds(sid * NPW, NPW)])


@functools.partial(
    pl.kernel,
    out_type=jax.ShapeDtypeStruct((NC, N, H), jnp.float32),
    mesh=_mesh,
    scratch_types=[
        pltpu.VMEM((NCHUNK, CH), jnp.int32),
        pltpu.VMEM((NCHUNK, CH), jnp.int32),
        pltpu.VMEM((KS, CH, H), jnp.float32),
        pltpu.VMEM((ZB, H), jnp.float32),
        pltpu.VMEM_SHARED((N, H), jnp.float32),
    ] + [pltpu.SemaphoreType.DMA] * (2 * KS),
    compiler_params=_sc_params,
)
def sc_scatter(h_hbm, src3_hbm, dst3_hbm, out_hbm,
               srci, dsti, rv, zbuf, acc, *sems):
    gsems = sems[:KS]
    ssems = sems[KS:]
    cid = lax.axis_index("c")
    sid = lax.axis_index("s")
    wid = sid * NC + cid

    zeros = jnp.zeros((16,), jnp.float32)

    pltpu.sync_copy(src3_hbm.at[wid], srci)
    pltpu.sync_copy(dst3_hbm.at[wid], dsti)

    @pl.when(sid < NF)
    def _init():
        def _zero(i, _):
            r = i // (H // 16)
            k = (i % (H // 16)) * 16
            zbuf[r, pl.ds(k, 16)] = zeros
            return 0

        lax.fori_loop(0, ZB * (H // 16), _zero, 0)
        for rr in range(NPW // ZB):
            pltpu.sync_copy(zbuf, acc.at[pl.ds(sid * NPW + rr * ZB, ZB)])

    plsc.subcore_barrier()

    def _round(t, _):
        for b in range(KS):
            i = t * KS + b

            @pl.when(i < NCHUNK)
            def _issue(i=i, b=b):
                pltpu.async_copy(h_hbm.at[srci.at[i]], rv.at[b], gsems[b])

            c = i - GLAG
            q = (b - GLAG) % KS

            @pl.when((c >= 0) & (c < NCHUNK))
            def _scat(c=c, q=q):
                pltpu.make_async_copy(
                    h_hbm.at[srci.at[c]], rv.at[q], gsems[q]).wait()

                @pl.when(c >= 1)
                def _wprev(c=c, q=q):
                    qp = (q - 1) % KS
                    pltpu.make_async_copy(
                        rv.at[qp], acc.at[dsti.at[c - 1]], ssems[0]).wait()

                pltpu.async_copy(rv.at[q], acc.at[dsti.at[c]], ssems[0],
                                 add=True)

        return 0

    lax.fori_loop(0, (NCHUNK + GLAG + KS - 1) // KS, _round, 0)
    pltpu.make_async_copy(rv.at[(NCHUNK - 1) % KS],
                          acc.at[dsti.at[NCHUNK - 1]], ssems[0]).wait()
    plsc.subcore_barrier()

    @pl.when(sid < NF)
    def _flush():
        pltpu.sync_copy(acc.at[pl.ds(sid * NPW, NPW)],
                        out_hbm.at[cid, pl.ds(sid * NPW, NPW)])


@functools.partial(
    pl.kernel,
    out_type=jax.ShapeDtypeStruct((E, H), jnp.float32),
    mesh=_mesh,
    scratch_types=[
        pltpu.VMEM((NCHUNK, CH), jnp.int32),
        pltpu.VMEM((NCHUNK, CH), jnp.int32),
        pltpu.VMEM((KE, CH, H), jnp.float32),
        pltpu.VMEM((KE, CH, H), jnp.float32),
        pltpu.VMEM((KE, CH, H), jnp.float32),
    ] + [pltpu.SemaphoreType.DMA] * (2 * KE),
    compiler_params=_sc_params,
)
def sc_gpair(atab, btab, src3_hbm, dst3_hbm, out_hbm,
             srci, dsti, av, bv, cv, *sems):
    cid = lax.axis_index("c")
    sid = lax.axis_index("s")
    wid = sid * NC + cid

    gsems = sems[:KE]
    ssems = sems[KE:]

    pltpu.sync_copy(src3_hbm.at[wid], srci)
    pltpu.sync_copy(dst3_hbm.at[wid], dsti)

    def _round(t, _):
        for b in range(KE):
            i = t * KE + b

            @pl.when(i >= KE)
            def _drain(i=i, b=b):
                base = wid * EPW + (i - KE) * CH
                pltpu.make_async_copy(
                    cv.at[b], out_hbm.at[pl.ds(base, CH)], ssems[b]).wait()

            @pl.when(i < NCHUNK)
            def _issue(i=i, b=b):
                pltpu.async_copy(atab.at[srci.at[i]], av.at[b], gsems[b])
                pltpu.async_copy(btab.at[dsti.at[i]], bv.at[b], gsems[b])

            c = i - GLAG
            q = (b - GLAG) % KE

            @pl.when((c >= 0) & (c < NCHUNK))
            def _compute(c=c, q=q):
                base = wid * EPW + c * CH
                pltpu.make_async_copy(
                    atab.at[srci.at[c]], av.at[q], gsems[q]).wait()
                pltpu.make_async_copy(
                    btab.at[dsti.at[c]], bv.at[q], gsems[q]).wait()

                def _ew(r, _, q=q):
                    for k in range(H // 16):
                        sl = pl.ds(k * 16, 16)
                        cv[q, r, sl] = av[q, r, sl] + bv[q, r, sl]
                    return 0

                lax.fori_loop(0, CH, _ew, 0, unroll=2)
                pltpu.async_copy(cv.at[q], out_hbm.at[pl.ds(base, CH)],
                                 ssems[q])

        return 0

    lax.fori_loop(0, (NCHUNK + KE + KE - 1) // KE, _round, 0)


def _full(shape):
    return pl.BlockSpec(shape, lambda i: tuple(0 for _ in shape))


def tc_node_encoder(x, Wn1, bn1, Wn2, bn2):
    blk = 1000

    def body(x_ref, w1_ref, b1_ref, w2_ref, b2_ref, out_ref):
        h = jnp.maximum(x_ref[...] @ w1_ref[...] + b1_ref[...], 0.0)
        out_ref[...] = h @ w2_ref[...] + b2_ref[...]

    return pl.pallas_call(
        body,
        grid=(N // blk,),
        in_specs=[pl.BlockSpec((blk, 128), lambda i: (i, 0)),
                  _full((128, 32)), _full((1, 32)),
                  _full((32, H)), _full((1, H))],
        out_specs=pl.BlockSpec((blk, H), lambda i: (i, 0)),
        out_shape=jax.ShapeDtypeStruct((N, H), jnp.float32),
    )(x, Wn1, bn1, Wn2, bn2)


def tc_edge_encoder(e, We1, be1, We2, be2):
    blk = 1600

    def body(e_ref, w1_ref, b1_ref, w2_ref, b2_ref, out_ref):
        h = jnp.maximum(e_ref[...] @ w1_ref[...] + b1_ref[...], 0.0)
        out_ref[...] = h @ w2_ref[...] + b2_ref[...]

    return pl.pallas_call(
        body,
        grid=(E // blk,),
        in_specs=[pl.BlockSpec((blk, 16), lambda i: (i, 0)),
                  _full((16, 32)), _full((1, 32)),
                  _full((32, H)), _full((1, H))],
        out_specs=pl.BlockSpec((blk, H), lambda i: (i, 0)),
        out_shape=jax.ShapeDtypeStruct((E, H), jnp.float32),
    )(e, We1, be1, We2, be2)


def tc_mlp_fused(g, eh, W, b):
    blk = 1600

    def body(g_ref, e_ref, w_ref, b_ref, out_ref):
        out_ref[...] = jnp.maximum(
            g_ref[...] + e_ref[...] @ w_ref[...] + b_ref[...], 0.0)

    return pl.pallas_call(
        body,
        grid=(E // blk,),
        in_specs=[pl.BlockSpec((blk, H), lambda i: (i, 0)),
                  pl.BlockSpec((blk, H), lambda i: (i, 0)),
                  _full((H, H)), _full((1, H))],
        out_specs=pl.BlockSpec((blk, H), lambda i: (i, 0)),
        out_shape=jax.ShapeDtypeStruct((E, H), jnp.float32),
    )(g, eh, W, b)


def tc_pred(g, eh, W, b1, wrow, bp2):
    blk = 512

    def body(g_ref, e_ref, w_ref, b1_ref, wr_ref, b2_ref, out_ref):
        t = jnp.maximum(g_ref[...] + e_ref[...] @ w_ref[...] + b1_ref[...],
                        0.0)
        out_ref[...] = jnp.sum(t * wr_ref[...], axis=1) + b2_ref[0, 0]

    return pl.pallas_call(
        body,
        grid=(E // blk,),
        in_specs=[pl.BlockSpec((blk, H), lambda i: (i, 0)),
                  pl.BlockSpec((blk, H), lambda i: (i, 0)),
                  _full((H, H)), _full((1, H)), _full((1, H)), _full((1, 1))],
        out_specs=pl.BlockSpec((blk,), lambda i: (i,)),
        out_shape=jax.ShapeDtypeStruct((E,), jnp.float32),
    )(g, eh, W, b1, wrow, bp2)


def tc_h_update(h, parts, degp, Wg_l, bg_l, W1, W2):
    blk = 1000

    def body(h_ref, p_ref, d_ref, wg_ref, bg_ref, w1_ref, w2_ref,
             hn_ref, a_ref, b_ref):
        h_ = h_ref[...]
        agg = p_ref[0] + p_ref[1]
        cnt = d_ref[0, :, :1] + d_ref[1, :, :1]
        inv = 1.0 / (cnt + 1.0)
        hn = jnp.maximum((h_ + (agg + h_) * inv) @ wg_ref[...] + bg_ref[...],
                         0.0)
        hn_ref[...] = hn
        a_ref[...] = hn @ w1_ref[...]
        b_ref[...] = hn @ w2_ref[...]

    return pl.pallas_call(
        body,
        grid=(N // blk,),
        in_specs=[pl.BlockSpec((blk, H), lambda i: (i, 0)),
                  pl.BlockSpec((NC, blk, H), lambda i: (0, i, 0)),
                  pl.BlockSpec((NC, blk, DW), lambda i: (0, i, 0)),
                  _full((H, H)), _full((1, H)), _full((H, H)), _full((H, H))],
        out_specs=[pl.BlockSpec((blk, H), lambda i: (i, 0)),
                   pl.BlockSpec((blk, H), lambda i: (i, 0)),
                   pl.BlockSpec((blk, H), lambda i: (i, 0))],
        out_shape=[jax.ShapeDtypeStruct((N, H), jnp.float32),
                   jax.ShapeDtypeStruct((N, H), jnp.float32),
                   jax.ShapeDtypeStruct((N, H), jnp.float32)],
    )(h, parts, degp, Wg_l, bg_l, W1, W2)


def tc_pair(h, P1, P2):
    blk = 1000

    def body(h_ref, w1_ref, w2_ref, a_ref, b_ref):
        h_ = h_ref[...]
        a_ref[...] = h_ @ w1_ref[...]
        b_ref[...] = h_ @ w2_ref[...]

    return pl.pallas_call(
        body,
        grid=(N // blk,),
        in_specs=[pl.BlockSpec((blk, H), lambda i: (i, 0)),
                  _full((H, H)), _full((H, H))],
        out_specs=[pl.BlockSpec((blk, H), lambda i: (i, 0)),
                   pl.BlockSpec((blk, H), lambda i: (i, 0))],
        out_shape=[jax.ShapeDtypeStruct((N, H), jnp.float32),
                   jax.ShapeDtypeStruct((N, H), jnp.float32)],
    )(h, P1, P2)


def kernel(x, e, edge_index, Wn1, bn1, Wn2, bn2, We1, be1, We2, be2,
           Wg, bg, Weg, beg, Wp1, bp1, Wp2, bp2):
    L = Wg.shape[0]
    src = edge_index[0]
    dst = edge_index[1]

    src3 = src.reshape(NW, NCHUNK, CH)
    dst3 = dst.reshape(NW, NCHUNK, CH)

    h = tc_node_encoder(x, Wn1, bn1.reshape(1, -1), Wn2, bn2.reshape(1, -1))
    eh = tc_edge_encoder(e, We1, be1.reshape(1, -1), We2, be2.reshape(1, -1))
    degp = sc_deg(dst.reshape(NW, NCHD, CHD))

    for l in range(L):
        W1 = Weg[l][:H]
        W2 = Weg[l][H:2 * H]
        W3 = Weg[l][2 * H:]
        parts = sc_scatter(h, src3, dst3)
        h, hs1, hs2 = tc_h_update(h, parts, degp, Wg[l], bg[l].reshape(1, -1),
                                  W1, W2)
        g = sc_gpair(hs1, hs2, src3, dst3)
        eh = tc_mlp_fused(g, eh, W3, beg[l].reshape(1, -1))

    hp1, hp2 = tc_pair(h, Wp1[:H], Wp1[H:2 * H])
    gp = sc_gpair(hp1, hp2, src3, dst3)
    return tc_pred(gp, eh, Wp1[2 * H:], bp1.reshape(1, -1),
                   Wp2.reshape(1, -1), bp2.reshape(1, 1))
